# Initial kernel scaffold; baseline (speedup 1.0000x reference)
import jax
import jax.numpy as jnp
from jax.experimental import pallas as pl


def kernel(nodes, features, edge_index, W, b):
    raise NotImplementedError("write your pallas kernel here")


# speedup vs baseline: 3.9127x; 3.9127x over previous
import jax
import jax.numpy as jnp
from jax import lax
from jax.experimental import pallas as pl
from jax.experimental.pallas import tpu as pltpu
from jax.experimental.pallas import tpu_sc as plsc

N = 10000
D = 128
H = D // 2
E = 320000
NC = 2
NS = 16
NW = NC * NS
CH = 80
CPT = E // (NS * CH)
ROWS_PT = 624


def _sc_body(ftab_hbm, src_hbm, dst_hbm, acc_out,
             acc_sh, idx_s, idx_d, idx_gs, idx_gd, rows, rows2, sem, sem2):
    c = lax.axis_index("c")
    s = lax.axis_index("s")

    def _zero_rows(i, _):
        for j in range(D // 16):
            rows[i, pl.ds(j * 16, 16)] = jnp.zeros((16,), jnp.float32)
        return 0

    lax.fori_loop(0, CH, _zero_rows, 0)

    base_r = s * ROWS_PT
    for k in range(ROWS_PT // CH):
        pltpu.sync_copy(rows, acc_sh.at[pl.ds(base_r + k * CH, CH)])
    rem = ROWS_PT - (ROWS_PT // CH) * CH
    pltpu.sync_copy(rows.at[pl.ds(0, rem)],
                    acc_sh.at[pl.ds(base_r + ROWS_PT - rem, rem)])

    @pl.when(s == NS - 1)
    def _zero_tail():
        tail = N - NS * ROWS_PT
        pltpu.sync_copy(rows.at[pl.ds(0, tail)],
                        acc_sh.at[pl.ds(NS * ROWS_PT, tail)])

    plsc.subcore_barrier()

    edge_base = s * (CPT * CH)
    coff = c * N

    def _chunk(k, _):
        base = edge_base + k * CH
        pltpu.sync_copy(src_hbm.at[pl.ds(base, CH)], idx_s)
        pltpu.sync_copy(dst_hbm.at[pl.ds(base, CH)], idx_d)
        for j in range(CH // 16):
            sl = pl.ds(j * 16, 16)
            idx_gs[sl] = idx_s[sl] + coff
            idx_gd[sl] = idx_d[sl] + coff
        cp1 = pltpu.async_copy(ftab_hbm.at[idx_gd], rows, sem)
        cp2 = pltpu.async_copy(ftab_hbm.at[idx_gs], rows2, sem2)
        cp1.wait()
        pltpu.sync_copy(rows, acc_sh.at[idx_s], add=True)
        cp2.wait()
        pltpu.sync_copy(rows2, acc_sh.at[idx_d], add=True)
        return 0

    lax.fori_loop(0, CPT, _chunk, 0)
    plsc.subcore_barrier()

    pltpu.sync_copy(acc_sh.at[pl.ds(base_r, ROWS_PT)],
                    acc_out.at[pl.ds(c * N + base_r, ROWS_PT)])

    @pl.when(s == NS - 1)
    def _write_tail():
        tail = N - NS * ROWS_PT
        pltpu.sync_copy(acc_sh.at[pl.ds(NS * ROWS_PT, tail)],
                        acc_out.at[pl.ds(c * N + NS * ROWS_PT, tail)])


def _sc_aggregate(ftab, src, dst):
    mesh = plsc.VectorSubcoreMesh(core_axis_name="c", subcore_axis_name="s")
    acc = pl.kernel(
        _sc_body,
        out_type=jax.ShapeDtypeStruct((NC * N, D), jnp.float32),
        mesh=mesh,
        scratch_types=[
            pltpu.VMEM_SHARED((N, D), jnp.float32),
            pltpu.VMEM((CH,), jnp.int32),
            pltpu.VMEM((CH,), jnp.int32),
            pltpu.VMEM((CH,), jnp.int32),
            pltpu.VMEM((CH,), jnp.int32),
            pltpu.VMEM((CH, D), jnp.float32),
            pltpu.VMEM((CH, D), jnp.float32),
            pltpu.SemaphoreType.DMA,
            pltpu.SemaphoreType.DMA,
        ],
    )(ftab, src, dst)
    return acc.reshape(NC, N, D)


def _tc_body(f_ref, acc_ref, w1_ref, w2_ref, b_ref, o_ref):
    sums = jnp.concatenate([acc_ref[0][:, :H], acc_ref[1][:, :H]], axis=1)
    deg = acc_ref[0][:, H:H + 1]
    neigh = sums / jnp.maximum(deg, 1.0)
    h = (jnp.dot(f_ref[...], w1_ref[...], preferred_element_type=jnp.float32)
         + jnp.dot(neigh, w2_ref[...], preferred_element_type=jnp.float32)
         + b_ref[...])
    o_ref[...] = jnp.maximum(h, 0.0)


def _tc_combine(features, acc, w1t, w2t, b):
    blk = 1000
    grid = (N // blk,)
    return pl.pallas_call(
        _tc_body,
        grid=grid,
        in_specs=[
            pl.BlockSpec((blk, D), lambda i: (i, 0)),
            pl.BlockSpec((NC, blk, D), lambda i: (0, i, 0)),
            pl.BlockSpec((D, D), lambda i: (0, 0)),
            pl.BlockSpec((D, D), lambda i: (0, 0)),
            pl.BlockSpec((1, D), lambda i: (0, 0)),
        ],
        out_specs=pl.BlockSpec((blk, D), lambda i: (i, 0)),
        out_shape=jax.ShapeDtypeStruct((N, D), jnp.float32),
    )(features, acc, w1t, w2t, b)


@jax.jit
def kernel(nodes, features, edge_index, W, b):
    src = edge_index[0]
    dst = edge_index[1]
    onecol = jnp.ones((N, 1), jnp.float32)
    zpad = jnp.zeros((N, H - 1), jnp.float32)
    ftab = jnp.concatenate([
        jnp.concatenate([features[:, :H], onecol, zpad], axis=1),
        jnp.concatenate([features[:, H:], onecol, zpad], axis=1),
    ], axis=0)
    acc = _sc_aggregate(ftab, src, dst)
    wt = W.T
    w1t = wt[:D]
    w2t = wt[D:]
    return _tc_combine(features, acc, w1t, w2t, b.reshape(1, -1))


# speedup vs baseline: 6.1209x; 1.5643x over previous
import jax
import jax.numpy as jnp
from jax import lax
from jax.experimental import pallas as pl
from jax.experimental.pallas import tpu as pltpu
from jax.experimental.pallas import tpu_sc as plsc

N = 10000
D = 128
H = D // 2
E = 320000
NC = 2
NS = 16
NW = NC * NS
CH = 80
CPT = E // (NS * CH)
ROWS_PT = 624


def _sc_body(ftab_hbm, src_hbm, dst_hbm, acc_out,
             acc_sh, idx_s, idx_d, idx_gs, idx_gd, rows, rows2,
             idx_s2, idx_d2, idx_gs2, idx_gd2, rowsB, rows2B,
             semA1, semA2, semB1, semB2):
    c = lax.axis_index("c")
    s = lax.axis_index("s")

    def _zero_rows(i, _):
        for j in range(D // 16):
            rows[i, pl.ds(j * 16, 16)] = jnp.zeros((16,), jnp.float32)
        return 0

    lax.fori_loop(0, CH, _zero_rows, 0)

    base_r = s * ROWS_PT
    for k in range(ROWS_PT // CH):
        pltpu.sync_copy(rows, acc_sh.at[pl.ds(base_r + k * CH, CH)])
    rem = ROWS_PT - (ROWS_PT // CH) * CH
    pltpu.sync_copy(rows.at[pl.ds(0, rem)],
                    acc_sh.at[pl.ds(base_r + ROWS_PT - rem, rem)])

    @pl.when(s == NS - 1)
    def _zero_tail():
        tail = N - NS * ROWS_PT
        pltpu.sync_copy(rows.at[pl.ds(0, tail)],
                        acc_sh.at[pl.ds(NS * ROWS_PT, tail)])

    plsc.subcore_barrier()

    edge_base = s * (CPT * CH)
    coff = c * N
    bufA = (idx_s, idx_d, idx_gs, idx_gd, rows, rows2)
    bufB = (idx_s2, idx_d2, idx_gs2, idx_gd2, rowsB, rows2B)

    def _issue(buf, smA, smB, base):
        b_s, b_d, b_gs, b_gd, b_r, b_r2 = buf
        pltpu.sync_copy(src_hbm.at[pl.ds(base, CH)], b_s)
        pltpu.sync_copy(dst_hbm.at[pl.ds(base, CH)], b_d)
        for j in range(CH // 16):
            sl = pl.ds(j * 16, 16)
            b_gs[sl] = b_s[sl] + coff
            b_gd[sl] = b_d[sl] + coff
        pltpu.async_copy(ftab_hbm.at[b_gd], b_r, smA)
        pltpu.async_copy(ftab_hbm.at[b_gs], b_r2, smB)

    def _drain_scatter(buf, smA, smB):
        b_s, b_d, b_gs, b_gd, b_r, b_r2 = buf
        pltpu.make_async_copy(ftab_hbm.at[b_gd], b_r, smA).wait()
        pltpu.sync_copy(b_r, acc_sh.at[b_s], add=True)
        pltpu.make_async_copy(ftab_hbm.at[b_gs], b_r2, smB).wait()
        pltpu.sync_copy(b_r2, acc_sh.at[b_d], add=True)

    _issue(bufA, semA1, semA2, edge_base)

    def _pair(k, _):
        _issue(bufB, semB1, semB2, edge_base + (2 * k + 1) * CH)
        _drain_scatter(bufA, semA1, semA2)

        @pl.when(k < CPT // 2 - 1)
        def _next_a():
            _issue(bufA, semA1, semA2, edge_base + (2 * k + 2) * CH)

        _drain_scatter(bufB, semB1, semB2)
        return 0

    lax.fori_loop(0, CPT // 2, _pair, 0)
    plsc.subcore_barrier()

    pltpu.sync_copy(acc_sh.at[pl.ds(base_r, ROWS_PT)],
                    acc_out.at[pl.ds(c * N + base_r, ROWS_PT)])

    @pl.when(s == NS - 1)
    def _write_tail():
        tail = N - NS * ROWS_PT
        pltpu.sync_copy(acc_sh.at[pl.ds(NS * ROWS_PT, tail)],
                        acc_out.at[pl.ds(c * N + NS * ROWS_PT, tail)])


def _sc_aggregate(ftab, src, dst):
    mesh = plsc.VectorSubcoreMesh(core_axis_name="c", subcore_axis_name="s")
    acc = pl.kernel(
        _sc_body,
        out_type=jax.ShapeDtypeStruct((NC * N, D), jnp.float32),
        mesh=mesh,
        scratch_types=[
            pltpu.VMEM_SHARED((N, D), jnp.float32),
            pltpu.VMEM((CH,), jnp.int32),
            pltpu.VMEM((CH,), jnp.int32),
            pltpu.VMEM((CH,), jnp.int32),
            pltpu.VMEM((CH,), jnp.int32),
            pltpu.VMEM((CH, D), jnp.float32),
            pltpu.VMEM((CH, D), jnp.float32),
            pltpu.VMEM((CH,), jnp.int32),
            pltpu.VMEM((CH,), jnp.int32),
            pltpu.VMEM((CH,), jnp.int32),
            pltpu.VMEM((CH,), jnp.int32),
            pltpu.VMEM((CH, D), jnp.float32),
            pltpu.VMEM((CH, D), jnp.float32),
            pltpu.SemaphoreType.DMA,
            pltpu.SemaphoreType.DMA,
            pltpu.SemaphoreType.DMA,
            pltpu.SemaphoreType.DMA,
        ],
    )(ftab, src, dst)
    return acc.reshape(NC, N, D)


def _tc_body(f_ref, acc_ref, w1_ref, w2_ref, b_ref, o_ref):
    sums = jnp.concatenate([acc_ref[0][:, :H], acc_ref[1][:, :H]], axis=1)
    deg = acc_ref[0][:, H:H + 1]
    neigh = sums / jnp.maximum(deg, 1.0)
    h = (jnp.dot(f_ref[...], w1_ref[...], preferred_element_type=jnp.float32)
         + jnp.dot(neigh, w2_ref[...], preferred_element_type=jnp.float32)
         + b_ref[...])
    o_ref[...] = jnp.maximum(h, 0.0)


def _tc_combine(features, acc, w1t, w2t, b):
    blk = 1000
    grid = (N // blk,)
    return pl.pallas_call(
        _tc_body,
        grid=grid,
        in_specs=[
            pl.BlockSpec((blk, D), lambda i: (i, 0)),
            pl.BlockSpec((NC, blk, D), lambda i: (0, i, 0)),
            pl.BlockSpec((D, D), lambda i: (0, 0)),
            pl.BlockSpec((D, D), lambda i: (0, 0)),
            pl.BlockSpec((1, D), lambda i: (0, 0)),
        ],
        out_specs=pl.BlockSpec((blk, D), lambda i: (i, 0)),
        out_shape=jax.ShapeDtypeStruct((N, D), jnp.float32),
    )(features, acc, w1t, w2t, b)


@jax.jit
def kernel(nodes, features, edge_index, W, b):
    src = edge_index[0]
    dst = edge_index[1]
    onecol = jnp.ones((N, 1), jnp.float32)
    zpad = jnp.zeros((N, H - 1), jnp.float32)
    ftab = jnp.concatenate([
        jnp.concatenate([features[:, :H], onecol, zpad], axis=1),
        jnp.concatenate([features[:, H:], onecol, zpad], axis=1),
    ], axis=0)
    acc = _sc_aggregate(ftab, src, dst)
    wt = W.T
    w1t = wt[:D]
    w2t = wt[D:]
    return _tc_combine(features, acc, w1t, w2t, b.reshape(1, -1))


# speedup vs baseline: 7.1814x; 1.1733x over previous
import jax
import jax.numpy as jnp
from jax import lax
from jax.experimental import pallas as pl
from jax.experimental.pallas import tpu as pltpu
from jax.experimental.pallas import tpu_sc as plsc

N = 10000
D = 128
H = D // 2
E = 320000
NC = 2
NS = 16
NW = NC * NS
CH = 80
EPT = E // NS
CPT = EPT // CH
ROWS_PT = 624
ZCH = 80


def _sc_body(ftab_hbm, src_hbm, dst_hbm, acc_out,
             acc_sh, idx_s, idx_d, idx_gs, idx_gd, rows, rows2,
             idx_s2, idx_d2, idx_gs2, idx_gd2, rowsB, rows2B,
             semA1, semA2, semB1, semB2, semI):
    c = lax.axis_index("c")
    s = lax.axis_index("s")

    def _zero_rows(i, _):
        for j in range(D // 16):
            rows[i, pl.ds(j * 16, 16)] = jnp.zeros((16,), jnp.float32)
        return 0

    lax.fori_loop(0, ZCH, _zero_rows, 0)

    base_r = s * ROWS_PT
    for k in range(ROWS_PT // ZCH):
        pltpu.sync_copy(rows.at[pl.ds(0, ZCH)],
                        acc_sh.at[pl.ds(base_r + k * ZCH, ZCH)])
    rem = ROWS_PT - (ROWS_PT // ZCH) * ZCH
    pltpu.sync_copy(rows.at[pl.ds(0, rem)],
                    acc_sh.at[pl.ds(base_r + ROWS_PT - rem, rem)])

    @pl.when(s == NS - 1)
    def _zero_tail():
        tail = N - NS * ROWS_PT
        pltpu.sync_copy(rows.at[pl.ds(0, tail)],
                        acc_sh.at[pl.ds(NS * ROWS_PT, tail)])

    plsc.subcore_barrier()

    edge_base = s * EPT
    coff = c * N
    bufA = (idx_s, idx_d, idx_gs, idx_gd, rows, rows2)
    bufB = (idx_s2, idx_d2, idx_gs2, idx_gd2, rowsB, rows2B)

    def _issue(buf, smA, smB, base):
        b_s, b_d, b_gs, b_gd, b_r, b_r2 = buf
        ci1 = pltpu.async_copy(src_hbm.at[pl.ds(base, CH)], b_s, semI)
        ci2 = pltpu.async_copy(dst_hbm.at[pl.ds(base, CH)], b_d, semI)
        ci1.wait()
        ci2.wait()
        for j in range(CH // 16):
            sl = pl.ds(j * 16, 16)
            b_gs[sl] = b_s[sl] + coff
            b_gd[sl] = b_d[sl] + coff
        pltpu.async_copy(ftab_hbm.at[b_gd], b_r, smA)
        pltpu.async_copy(ftab_hbm.at[b_gs], b_r2, smB)

    def _drain_scatter(buf, smA, smB):
        b_s, b_d, b_gs, b_gd, b_r, b_r2 = buf
        pltpu.make_async_copy(ftab_hbm.at[b_gd], b_r, smA).wait()
        pltpu.sync_copy(b_r, acc_sh.at[b_s], add=True)
        pltpu.make_async_copy(ftab_hbm.at[b_gs], b_r2, smB).wait()
        pltpu.sync_copy(b_r2, acc_sh.at[b_d], add=True)

    _issue(bufA, semA1, semA2, edge_base)

    def _pair(k, _):
        _issue(bufB, semB1, semB2, edge_base + (2 * k + 1) * CH)
        _drain_scatter(bufA, semA1, semA2)

        @pl.when(k < CPT // 2 - 1)
        def _next_a():
            _issue(bufA, semA1, semA2, edge_base + (2 * k + 2) * CH)

        _drain_scatter(bufB, semB1, semB2)
        return 0

    lax.fori_loop(0, CPT // 2, _pair, 0)
    plsc.subcore_barrier()

    pltpu.sync_copy(acc_sh.at[pl.ds(base_r, ROWS_PT)],
                    acc_out.at[pl.ds(c * N + base_r, ROWS_PT)])

    @pl.when(s == NS - 1)
    def _write_tail():
        tail = N - NS * ROWS_PT
        pltpu.sync_copy(acc_sh.at[pl.ds(NS * ROWS_PT, tail)],
                        acc_out.at[pl.ds(c * N + NS * ROWS_PT, tail)])


def _sc_aggregate(ftab, src, dst):
    mesh = plsc.VectorSubcoreMesh(core_axis_name="c", subcore_axis_name="s")
    acc = pl.kernel(
        _sc_body,
        out_type=jax.ShapeDtypeStruct((NC * N, D), jnp.float32),
        mesh=mesh,
        scratch_types=[
            pltpu.VMEM_SHARED((N, D), jnp.float32),
            pltpu.VMEM((CH,), jnp.int32),
            pltpu.VMEM((CH,), jnp.int32),
            pltpu.VMEM((CH,), jnp.int32),
            pltpu.VMEM((CH,), jnp.int32),
            pltpu.VMEM((CH, D), jnp.float32),
            pltpu.VMEM((CH, D), jnp.float32),
            pltpu.VMEM((CH,), jnp.int32),
            pltpu.VMEM((CH,), jnp.int32),
            pltpu.VMEM((CH,), jnp.int32),
            pltpu.VMEM((CH,), jnp.int32),
            pltpu.VMEM((CH, D), jnp.float32),
            pltpu.VMEM((CH, D), jnp.float32),
            pltpu.SemaphoreType.DMA,
            pltpu.SemaphoreType.DMA,
            pltpu.SemaphoreType.DMA,
            pltpu.SemaphoreType.DMA,
            pltpu.SemaphoreType.DMA,
        ],
    )(ftab, src, dst)
    return acc.reshape(NC, N, D)


def _tc_body(f_ref, acc_ref, w1_ref, w2_ref, b_ref, o_ref):
    sums = jnp.concatenate([acc_ref[0][:, :H], acc_ref[1][:, :H]], axis=1)
    deg = acc_ref[0][:, H:H + 1]
    neigh = sums / jnp.maximum(deg, 1.0)
    h = (jnp.dot(f_ref[...], w1_ref[...], preferred_element_type=jnp.float32)
         + jnp.dot(neigh, w2_ref[...], preferred_element_type=jnp.float32)
         + b_ref[...])
    o_ref[...] = jnp.maximum(h, 0.0)


def _tc_combine(features, acc, w1t, w2t, b):
    blk = 1000
    grid = (N // blk,)
    return pl.pallas_call(
        _tc_body,
        grid=grid,
        in_specs=[
            pl.BlockSpec((blk, D), lambda i: (i, 0)),
            pl.BlockSpec((NC, blk, D), lambda i: (0, i, 0)),
            pl.BlockSpec((D, D), lambda i: (0, 0)),
            pl.BlockSpec((D, D), lambda i: (0, 0)),
            pl.BlockSpec((1, D), lambda i: (0, 0)),
        ],
        out_specs=pl.BlockSpec((blk, D), lambda i: (i, 0)),
        out_shape=jax.ShapeDtypeStruct((N, D), jnp.float32),
    )(features, acc, w1t, w2t, b)


@jax.jit
def kernel(nodes, features, edge_index, W, b):
    src = edge_index[0]
    dst = edge_index[1]
    onecol = jnp.ones((N, 1), jnp.float32)
    zpad = jnp.zeros((N, H - 1), jnp.float32)
    ftab = jnp.concatenate([
        jnp.concatenate([features[:, :H], onecol, zpad], axis=1),
        jnp.concatenate([features[:, H:], onecol, zpad], axis=1),
    ], axis=0)
    acc = _sc_aggregate(ftab, src, dst)
    wt = W.T
    w1t = wt[:D]
    w2t = wt[D:]
    return _tc_combine(features, acc, w1t, w2t, b.reshape(1, -1))
